# baseline (speedup 1.0000x reference)
import functools

import jax
import jax.numpy as jnp
from jax import lax
from jax.experimental import pallas as pl
from jax.experimental.pallas import tpu as pltpu
from jax.experimental.pallas import tpu_sc as plsc

_NC = 2
_NS = 16
_NW = _NC * _NS
_L = 16


def _matmul_bias_kernel(x_ref, w_ref, b_ref, o_ref):
    o_ref[...] = (
        jnp.dot(x_ref[...], w_ref[...], preferred_element_type=jnp.float32)
        + b_ref[...]
    )


def _emb_kernel(xt_ref, w_ref, b_ref, nrm_ref, o_ref):
    embf = lax.dot_general(
        xt_ref[...], w_ref[...], (((0,), (0,)), ((), ())),
        preferred_element_type=jnp.float32) + b_ref[...]
    r = lax.bitcast_convert_type(embf, jnp.int32)
    t = r + 0x7FFF + ((r >> 16) & 1)
    hb = (t >> 16) & 0xFFFF
    lo = jnp.concatenate(
        [hb[:, 32 * g:32 * g + 16] for g in range(4)], axis=1)
    hi = jnp.concatenate(
        [hb[:, 32 * g + 16:32 * g + 32] for g in range(4)], axis=1)
    ncol = jnp.transpose(nrm_ref[...], (1, 0))
    nb = jnp.broadcast_to(ncol, (embf.shape[0], _L))
    nbits = lax.bitcast_convert_type(nb, jnp.int32)
    o_ref[...] = jnp.concatenate([lo | (hi << 16), nbits], axis=1)


def _edge_emb(ef_t, w_edge, b_edge, norm_r, block_rows):
    k, m = ef_t.shape
    _, dout = w_edge.shape
    assert m % block_rows == 0 and block_rows % 128 == 0
    return pl.pallas_call(
        _emb_kernel,
        grid=(m // block_rows,),
        in_specs=[
            pl.BlockSpec((k, block_rows), lambda i: (0, i)),
            pl.BlockSpec((k, dout), lambda i: (0, 0)),
            pl.BlockSpec((1, dout), lambda i: (0, 0)),
            pl.BlockSpec((1, block_rows), lambda i: (0, i)),
        ],
        out_specs=pl.BlockSpec((block_rows, dout // 2 + _L),
                               lambda i: (i, 0)),
        out_shape=jax.ShapeDtypeStruct((m, dout // 2 + _L), jnp.int32),
    )(ef_t, w_edge, b_edge.reshape(1, dout), norm_r)


def _matmul_bias(x, w, b, block_rows):
    m, k = x.shape
    _, dout = w.shape
    assert m % block_rows == 0
    return pl.pallas_call(
        _matmul_bias_kernel,
        grid=(m // block_rows,),
        in_specs=[
            pl.BlockSpec((block_rows, k), lambda i: (i, 0)),
            pl.BlockSpec((k, dout), lambda i: (0, 0)),
            pl.BlockSpec((1, dout), lambda i: (0, 0)),
        ],
        out_specs=pl.BlockSpec((block_rows, dout), lambda i: (i, 0)),
        out_shape=jax.ShapeDtypeStruct((m, dout), jnp.float32),
    )(x, w, b.reshape(1, dout))


def _combine_kernel(p0_ref, p1_ref, h_ref, d_ref, r_ref, o_ref):
    self_term = jnp.maximum(h_ref[...] + r_ref[...], 0.0) / d_ref[...]
    o_ref[...] = p0_ref[0] + p1_ref[0] + self_term


def _combine(partial, h, degs2d, root_w, block_rows):
    n, d = h.shape
    assert n % block_rows == 0
    return pl.pallas_call(
        _combine_kernel,
        grid=(n // block_rows,),
        in_specs=[
            pl.BlockSpec((1, block_rows, d), lambda i: (0, i, 0)),
            pl.BlockSpec((1, block_rows, d), lambda i: (1, i, 0)),
            pl.BlockSpec((block_rows, d), lambda i: (i, 0)),
            pl.BlockSpec((block_rows, 1), lambda i: (i, 0)),
            pl.BlockSpec((1, d), lambda i: (0, 0)),
        ],
        out_specs=pl.BlockSpec((block_rows, d), lambda i: (i, 0)),
        out_shape=jax.ShapeDtypeStruct((n, d), jnp.float32),
    )(partial, partial, h, degs2d, root_w)


def _sc_edge_aggregate(h, emb, src, dst):
    n, d = h.shape
    e = src.shape[0]
    C = 40
    ew = e // _NW
    n_chunks = ew // C
    assert ew % C == 0 and e % _NW == 0
    BR = 40
    nblk = n // BR
    assert n % BR == 0
    njd = d // _L

    NB = 3
    n_groups = (n_chunks + NB - 1) // NB

    mesh = plsc.VectorSubcoreMesh(core_axis_name="c", subcore_axis_name="s")

    PW = d // 2 + _L
    scratch = (
        [pltpu.VMEM((C,), jnp.int32) for _ in range(NB)]
        + [pltpu.VMEM((C,), jnp.int32) for _ in range(NB)]
        + [pltpu.VMEM((C, d), jnp.float32) for _ in range(NB)]
        + [pltpu.VMEM((C, PW), jnp.int32) for _ in range(NB)]
        + [pltpu.VMEM_SHARED((n, d), jnp.float32)]
        + [pltpu.SemaphoreType.DMA for _ in range(3 * NB)]
    )

    @functools.partial(
        pl.kernel,
        out_type=jax.ShapeDtypeStruct((_NC, n, d), jnp.float32),
        mesh=mesh,
        scratch_types=scratch,
    )
    def k(h_hbm, emb_hbm, src_hbm, dst_hbm, out_hbm, *refs):
        src_v = refs[0:NB]
        dst_v = refs[NB:2 * NB]
        rows_v = refs[2 * NB:3 * NB]
        emb_v = refs[3 * NB:4 * NB]
        zero_v = refs[2 * NB]
        acc_sh = refs[4 * NB]
        in_sem = refs[4 * NB + 1:4 * NB + 1 + NB]
        gat_sem = refs[4 * NB + 1 + NB:4 * NB + 1 + 2 * NB]
        sct_sem = refs[4 * NB + 1 + 2 * NB:4 * NB + 1 + 3 * NB]

        cid = lax.axis_index("c")
        sid = lax.axis_index("s")

        def zrow(r, carry):
            for j in range(njd):
                zero_v[r, pl.ds(j * _L, _L)] = jnp.zeros((_L,), jnp.float32)
            return carry
        lax.fori_loop(0, BR, zrow, 0)
        my_blocks = nblk // _NS + jnp.where(sid < nblk % _NS, 1, 0)

        def zblk(b, carry):
            row = (b * _NS + sid) * BR
            pltpu.sync_copy(zero_v, acc_sh.at[pl.ds(row, BR), :])
            return carry
        lax.fori_loop(0, my_blocks, zblk, 0)
        plsc.subcore_barrier()

        wid = cid * _NS + sid

        def fire_inputs(ci, b):
            ebase = (ci * _NW + wid) * C
            pltpu.async_copy(src_hbm.at[pl.ds(ebase, C)], src_v[b], in_sem[b])
            pltpu.async_copy(dst_hbm.at[pl.ds(ebase, C)], dst_v[b], in_sem[b])
            pltpu.async_copy(emb_hbm.at[pl.ds(ebase, C), :], emb_v[b],
                             in_sem[b])

        def wait_inputs(b):
            pltpu.make_async_copy(src_hbm.at[pl.ds(0, C)], src_v[b],
                                  in_sem[b]).wait()
            pltpu.make_async_copy(dst_hbm.at[pl.ds(0, C)], dst_v[b],
                                  in_sem[b]).wait()
            pltpu.make_async_copy(emb_hbm.at[pl.ds(0, C), :], emb_v[b],
                                  in_sem[b]).wait()

        def fire_gather(b):
            pltpu.async_copy(h_hbm.at[src_v[b]], rows_v[b], gat_sem[b])

        def wait_scatter(b):
            pltpu.make_async_copy(rows_v[b], acc_sh.at[dst_v[b]],
                                  sct_sem[b]).wait()

        for b in range(NB - 1):
            fire_inputs(b, b)
        wait_inputs(0)
        fire_gather(0)

        def group(g, carry):
            for b in range(NB):
                ci = g * NB + b
                bpre = (b + NB - 1) % NB
                jc = ci + NB - 1

                @pl.when(jc < n_chunks)
                def _():
                    @pl.when(jc >= NB)
                    def _():
                        wait_scatter(bpre)
                    fire_inputs(jc, bpre)

                @pl.when(ci < n_chunks)
                def _():
                    pltpu.make_async_copy(h_hbm.at[src_v[b]], rows_v[b],
                                          gat_sem[b]).wait()

                    def edge(ei, ecarry):
                        nv = lax.bitcast_convert_type(
                            emb_v[b][ei, pl.ds(d // 2, _L)], jnp.float32)
                        for j2 in range(njd // 2):
                            w16 = emb_v[b][ei, pl.ds(j2 * _L, _L)]
                            m0 = lax.bitcast_convert_type(
                                w16 << 16, jnp.float32)
                            m1 = lax.bitcast_convert_type(
                                w16 & jnp.int32(-65536), jnp.float32)
                            sl0 = pl.ds(j2 * 2 * _L, _L)
                            sl1 = pl.ds((j2 * 2 + 1) * _L, _L)
                            rows_v[b][ei, sl0] = jnp.maximum(
                                rows_v[b][ei, sl0] + m0, 0.0) * nv
                            rows_v[b][ei, sl1] = jnp.maximum(
                                rows_v[b][ei, sl1] + m1, 0.0) * nv
                        return ecarry
                    lax.fori_loop(0, C, edge, 0)

                    pltpu.async_copy(rows_v[b], acc_sh.at[dst_v[b]],
                                     sct_sem[b], add=True)

                    bnx = (b + 1) % NB

                    @pl.when(ci + 1 < n_chunks)
                    def _():
                        wait_inputs(bnx)
                        fire_gather(bnx)
            return carry
        lax.fori_loop(0, n_groups, group, 0)

        for b in range(NB):
            wait_scatter(b)
        plsc.subcore_barrier()

        def wblk(b, carry):
            row = (b * _NS + sid) * BR
            pltpu.sync_copy(acc_sh.at[pl.ds(row, BR), :],
                            out_hbm.at[cid, pl.ds(row, BR), :])
            return carry
        lax.fori_loop(0, my_blocks, wblk, 0)

    return k(h, emb, src, dst)


def kernel(nfeat, efeat, degs, norm, edge_index, W_lin, b_lin, W_edge, b_edge,
           root_w):
    n, d = nfeat.shape

    h = _matmul_bias(nfeat, W_lin, b_lin, block_rows=1000)
    e = efeat.shape[0]
    emb = _edge_emb(efeat.T, W_edge, b_edge,
                    norm.reshape(1, e), block_rows=6400)

    src = edge_index[0]
    dst = edge_index[1]
    partial = _sc_edge_aggregate(h, emb, src, dst)

    return _combine(partial, h, degs.reshape(n, 1), root_w, block_rows=1000)


# speedup vs baseline: 1.2798x; 1.2798x over previous
import functools

import jax
import jax.numpy as jnp
from jax import lax
from jax.experimental import pallas as pl
from jax.experimental.pallas import tpu as pltpu
from jax.experimental.pallas import tpu_sc as plsc

_NC = 2
_NS = 16
_NW = _NC * _NS
_L = 16


def _matmul_bias_kernel(x_ref, w_ref, b_ref, o_ref):
    o_ref[...] = (
        jnp.dot(x_ref[...], w_ref[...], preferred_element_type=jnp.float32)
        + b_ref[...]
    )


def _emb_kernel(xt_ref, w_ref, b_ref, nrm_ref, o_ref, nb_ref):
    o_ref[...] = lax.dot_general(
        xt_ref[...], w_ref[...], (((0,), (0,)), ((), ())),
        preferred_element_type=jnp.float32) + b_ref[...]
    ncol = jnp.transpose(nrm_ref[...], (1, 0))
    nb_ref[...] = jnp.broadcast_to(ncol, nb_ref.shape)


def _edge_emb(ef_t, w_edge, b_edge, norm_r, block_rows):
    k, m = ef_t.shape
    _, dout = w_edge.shape
    assert m % block_rows == 0 and block_rows % 128 == 0
    return pl.pallas_call(
        _emb_kernel,
        grid=(m // block_rows,),
        in_specs=[
            pl.BlockSpec((k, block_rows), lambda i: (0, i)),
            pl.BlockSpec((k, dout), lambda i: (0, 0)),
            pl.BlockSpec((1, dout), lambda i: (0, 0)),
            pl.BlockSpec((1, block_rows), lambda i: (0, i)),
        ],
        out_specs=[
            pl.BlockSpec((block_rows, dout), lambda i: (i, 0)),
            pl.BlockSpec((block_rows, _L), lambda i: (i, 0)),
        ],
        out_shape=[
            jax.ShapeDtypeStruct((m, dout), jnp.float32),
            jax.ShapeDtypeStruct((m, _L), jnp.float32),
        ],
    )(ef_t, w_edge, b_edge.reshape(1, dout), norm_r)


def _matmul_bias(x, w, b, block_rows):
    m, k = x.shape
    _, dout = w.shape
    assert m % block_rows == 0
    return pl.pallas_call(
        _matmul_bias_kernel,
        grid=(m // block_rows,),
        in_specs=[
            pl.BlockSpec((block_rows, k), lambda i: (i, 0)),
            pl.BlockSpec((k, dout), lambda i: (0, 0)),
            pl.BlockSpec((1, dout), lambda i: (0, 0)),
        ],
        out_specs=pl.BlockSpec((block_rows, dout), lambda i: (i, 0)),
        out_shape=jax.ShapeDtypeStruct((m, dout), jnp.float32),
    )(x, w, b.reshape(1, dout))


def _combine_kernel(p0_ref, p1_ref, h_ref, d_ref, r_ref, o_ref):
    self_term = jnp.maximum(h_ref[...] + r_ref[...], 0.0) / d_ref[...]
    o_ref[...] = p0_ref[0] + p1_ref[0] + self_term


def _combine(partial, h, degs2d, root_w, block_rows):
    n, d = h.shape
    assert n % block_rows == 0
    return pl.pallas_call(
        _combine_kernel,
        grid=(n // block_rows,),
        in_specs=[
            pl.BlockSpec((1, block_rows, d), lambda i: (0, i, 0)),
            pl.BlockSpec((1, block_rows, d), lambda i: (1, i, 0)),
            pl.BlockSpec((block_rows, d), lambda i: (i, 0)),
            pl.BlockSpec((block_rows, 1), lambda i: (i, 0)),
            pl.BlockSpec((1, d), lambda i: (0, 0)),
        ],
        out_specs=pl.BlockSpec((block_rows, d), lambda i: (i, 0)),
        out_shape=jax.ShapeDtypeStruct((n, d), jnp.float32),
    )(partial, partial, h, degs2d, root_w)


def _sc_edge_aggregate(h, emb, normb, src, dst):
    n, d = h.shape
    e = src.shape[0]
    C = 40
    ew = e // _NW
    n_chunks = ew // C
    assert ew % C == 0 and e % _NW == 0
    BR = 40
    nblk = n // BR
    assert n % BR == 0
    njd = d // _L

    NB = 3
    n_groups = (n_chunks + NB - 1) // NB

    mesh = plsc.VectorSubcoreMesh(core_axis_name="c", subcore_axis_name="s")

    scratch = (
        [pltpu.VMEM((C,), jnp.int32) for _ in range(NB)]
        + [pltpu.VMEM((C,), jnp.int32) for _ in range(NB)]
        + [pltpu.VMEM((C, _L), jnp.float32) for _ in range(NB)]
        + [pltpu.VMEM((C, d), jnp.float32) for _ in range(NB)]
        + [pltpu.VMEM((C, d), jnp.float32) for _ in range(NB)]
        + [pltpu.VMEM_SHARED((n, d), jnp.float32)]
        + [pltpu.SemaphoreType.DMA for _ in range(3 * NB)]
    )

    @functools.partial(
        pl.kernel,
        out_type=jax.ShapeDtypeStruct((_NC, n, d), jnp.float32),
        mesh=mesh,
        scratch_types=scratch,
    )
    def k(h_hbm, emb_hbm, norm_hbm, src_hbm, dst_hbm, out_hbm, *refs):
        src_v = refs[0:NB]
        dst_v = refs[NB:2 * NB]
        norm_v = refs[2 * NB:3 * NB]
        rows_v = refs[3 * NB:4 * NB]
        emb_v = refs[4 * NB:5 * NB]
        zero_v = refs[3 * NB]
        acc_sh = refs[5 * NB]
        in_sem = refs[5 * NB + 1:5 * NB + 1 + NB]
        gat_sem = refs[5 * NB + 1 + NB:5 * NB + 1 + 2 * NB]
        sct_sem = refs[5 * NB + 1 + 2 * NB:5 * NB + 1 + 3 * NB]

        cid = lax.axis_index("c")
        sid = lax.axis_index("s")

        def zrow(r, carry):
            for j in range(njd):
                zero_v[r, pl.ds(j * _L, _L)] = jnp.zeros((_L,), jnp.float32)
            return carry
        lax.fori_loop(0, BR, zrow, 0)
        my_blocks = nblk // _NS + jnp.where(sid < nblk % _NS, 1, 0)

        def zblk(b, carry):
            row = (b * _NS + sid) * BR
            pltpu.sync_copy(zero_v, acc_sh.at[pl.ds(row, BR), :])
            return carry
        lax.fori_loop(0, my_blocks, zblk, 0)
        plsc.subcore_barrier()

        wid = cid * _NS + sid

        def fire_inputs(ci, b):
            ebase = (ci * _NW + wid) * C
            pltpu.async_copy(src_hbm.at[pl.ds(ebase, C)], src_v[b], in_sem[b])
            pltpu.async_copy(dst_hbm.at[pl.ds(ebase, C)], dst_v[b], in_sem[b])
            pltpu.async_copy(norm_hbm.at[pl.ds(ebase, C), :], norm_v[b],
                             in_sem[b])
            pltpu.async_copy(emb_hbm.at[pl.ds(ebase, C), :], emb_v[b],
                             in_sem[b])

        def wait_inputs(b):
            pltpu.make_async_copy(src_hbm.at[pl.ds(0, C)], src_v[b],
                                  in_sem[b]).wait()
            pltpu.make_async_copy(dst_hbm.at[pl.ds(0, C)], dst_v[b],
                                  in_sem[b]).wait()
            pltpu.make_async_copy(norm_hbm.at[pl.ds(0, C), :], norm_v[b],
                                  in_sem[b]).wait()
            pltpu.make_async_copy(emb_hbm.at[pl.ds(0, C), :], emb_v[b],
                                  in_sem[b]).wait()

        def fire_gather(b):
            pltpu.async_copy(h_hbm.at[src_v[b]], rows_v[b], gat_sem[b])

        def wait_scatter(b):
            pltpu.make_async_copy(rows_v[b], acc_sh.at[dst_v[b]],
                                  sct_sem[b]).wait()

        for b in range(NB - 1):
            fire_inputs(b, b)
        wait_inputs(0)
        fire_gather(0)

        def group(g, carry):
            for b in range(NB):
                ci = g * NB + b
                bpre = (b + NB - 1) % NB
                jc = ci + NB - 1

                @pl.when(jc < n_chunks)
                def _():
                    @pl.when(jc >= NB)
                    def _():
                        wait_scatter(bpre)
                    fire_inputs(jc, bpre)

                @pl.when(ci < n_chunks)
                def _():
                    pltpu.make_async_copy(h_hbm.at[src_v[b]], rows_v[b],
                                          gat_sem[b]).wait()

                    def edge(ei, ecarry):
                        nv = norm_v[b][ei, :]
                        for j in range(njd):
                            sl = pl.ds(j * _L, _L)
                            rows_v[b][ei, sl] = jnp.maximum(
                                rows_v[b][ei, sl] + emb_v[b][ei, sl], 0.0) * nv
                        return ecarry
                    lax.fori_loop(0, C, edge, 0)

                    pltpu.async_copy(rows_v[b], acc_sh.at[dst_v[b]],
                                     sct_sem[b], add=True)

                    bnx = (b + 1) % NB

                    @pl.when(ci + 1 < n_chunks)
                    def _():
                        wait_inputs(bnx)
                        fire_gather(bnx)
            return carry
        lax.fori_loop(0, n_groups, group, 0)

        for b in range(NB):
            wait_scatter(b)
        plsc.subcore_barrier()

        def wblk(b, carry):
            row = (b * _NS + sid) * BR
            pltpu.sync_copy(acc_sh.at[pl.ds(row, BR), :],
                            out_hbm.at[cid, pl.ds(row, BR), :])
            return carry
        lax.fori_loop(0, my_blocks, wblk, 0)

    return k(h, emb, normb, src, dst)


def kernel(nfeat, efeat, degs, norm, edge_index, W_lin, b_lin, W_edge, b_edge,
           root_w):
    n, d = nfeat.shape

    h = _matmul_bias(nfeat, W_lin, b_lin, block_rows=1000)
    e = efeat.shape[0]
    emb, normb = _edge_emb(efeat.T, W_edge, b_edge,
                           norm.reshape(1, e), block_rows=6400)

    src = edge_index[0]
    dst = edge_index[1]
    partial = _sc_edge_aggregate(h, emb, normb, src, dst)

    return _combine(partial, h, degs.reshape(n, 1), root_w, block_rows=1000)
